# baseline (speedup 1.0000x reference)
import functools

import jax
import jax.numpy as jnp
from jax import lax
from jax.experimental import pallas as pl
from jax.experimental.pallas import tpu as pltpu
from jax.experimental.pallas import tpu_sc as plsc

N = 10000
D = 128
E = 160000
NE = 5000
NTILES = 16
EPT = E // NTILES
CHUNK = 80
NCHUNK = EPT // CHUNK
ACC_ROWS = 5120
ROWS_PT = ACC_ROWS // NTILES


def _zero16():
    return jnp.zeros((16,), jnp.float32)


NCB = 6
NGB = 3
NGRP = CHUNK // 16


def _spmm_body(ebs_hbm, comb_hbm, out_hbm,
               acc, cb0, cb1, cb2, cb3, cb4, cb5, gbuf0, gbuf1, gbuf2,
               cs0, cs1, cs2, cs3, cs4, cs5, gs0, gs1, gs2, ss0, ss1, ss2):
    c = lax.axis_index("c")
    s = lax.axis_index("s")
    row0 = s * ROWS_PT
    cbufs = [cb0, cb1, cb2, cb3, cb4, cb5]
    csems = [cs0, cs1, cs2, cs3, cs4, cs5]
    gbufs = [gbuf0, gbuf1, gbuf2]
    gsems = [gs0, gs1, gs2]
    ssems = [ss0, ss1, ss2]

    def zrow(r, _):
        for j in range(8):
            gbuf0[r, pl.ds(j * 16, 16)] = _zero16()
        return 0
    lax.fori_loop(0, CHUNK, zrow, 0)
    for m in range(2):
        for q in range(ROWS_PT // CHUNK):
            pltpu.sync_copy(gbuf0, acc.at[m, pl.ds(row0 + q * CHUNK, CHUNK)])
    plsc.subcore_barrier()

    def start_cload(m, k, b):
        pltpu.async_copy(comb_hbm.at[c, m, s, k], cbufs[b], csems[b])

    def wait_cload(b):
        pltpu.make_async_copy(comb_hbm.at[0, 0, 0, 0], cbufs[b], csems[b]).wait()

    def start_gather(b6, b3):
        pltpu.async_copy(ebs_hbm.at[pl.ds(0, CHUNK)], gbufs[b3], gsems[b3])

    def wait_gather(b3):
        pltpu.make_async_copy(ebs_hbm.at[pl.ds(0, CHUNK)], gbufs[b3],
                              gsems[b3]).wait()

    def start_scatter(m, b3, b6):
        pltpu.async_copy(gbufs[b3], acc.at[m, pl.ds(0, CHUNK)], ssems[b3])

    def wait_scatter(b3):
        pltpu.make_async_copy(ebs_hbm.at[pl.ds(0, CHUNK)], gbufs[b3],
                              ssems[b3]).wait()

    def scale(b3, b6):
        buf = gbufs[b3]
        cb = cbufs[b6]

        def grp(g, _):
            vv = plsc.bitcast(cb[2, pl.ds(g * 16, 16)], jnp.float32)
            for t in range(16):
                vb = vv.at[jnp.full((16,), t, jnp.int32)].get(
                    mode='promise_in_bounds')
                r = g * 16 + t
                for jj in range(8):
                    sl = pl.ds(jj * 16, 16)
                    buf[r, sl] = buf[r, sl] * vb
            return 0
        pass

    def step(m, k, j, maybe_first=False, launch=True, refill=True):
        b3, b6 = j % NGB, j % NCB
        wait_gather(b3)
        scale(b3, b6)
        start_scatter(m, b3, b6)
        if launch:
            b3n, b6n = (j + 2) % NGB, (j + 2) % NCB

            def _refill():
                if refill:
                    start_cload(m, k + NCB - 1, (j + NCB - 1) % NCB)

            if maybe_first:
                @pl.when(k >= 1)
                def _():
                    wait_scatter(b3n)
                    _refill()

                @pl.when(k < 1)
                def _():
                    _refill()
            else:
                wait_scatter(b3n)
                _refill()
            wait_cload(b6n)
            start_gather(b6n, b3n)

    def matrix_body(m, _):
        for b in range(NGB + 2):
            start_cload(m, b, b)
        wait_cload(0)
        start_gather(0, 0)
        wait_cload(1)
        start_gather(1, 1)

        def six(i, _):
            for j in range(NCB):
                step(m, i * NCB + j, j, maybe_first=(j == 0))
            return 0

        lax.fori_loop(0, (NCHUNK - 5) // NCB, six, 0)
        for j in range(5):
            k = NCHUNK - 5 + j
            step(m, k, k % NCB, launch=(k + 2 < NCHUNK),
                 refill=(k + NCB - 1 < NCHUNK))
        for k in range(NCHUNK - 3, NCHUNK):
            wait_scatter(k % NGB)
        return 0

    lax.fori_loop(0, 2, matrix_body, 0)

    plsc.subcore_barrier()

    for m in range(2):
        @pl.when(s < NTILES - 1)
        def _():
            pltpu.sync_copy(acc.at[m, pl.ds(row0, ROWS_PT)],
                            out_hbm.at[m, c, pl.ds(row0, ROWS_PT)])

        @pl.when(s == NTILES - 1)
        def _():
            pltpu.sync_copy(acc.at[m, pl.ds(NE - 200, 200)],
                            out_hbm.at[m, c, pl.ds(NE - 200, 200)])


_spmm = pl.kernel(
    _spmm_body,
    out_type=jax.ShapeDtypeStruct((2, 2, NE, D), jnp.float32),
    mesh=plsc.VectorSubcoreMesh(core_axis_name="c", subcore_axis_name="s"),
    compiler_params=pltpu.CompilerParams(needs_layout_passes=False),
    scratch_types=(
        [pltpu.VMEM_SHARED((2, ACC_ROWS, D), jnp.float32)]
        + [pltpu.VMEM((3, CHUNK), jnp.int32) for _ in range(NCB)]
        + [pltpu.VMEM((CHUNK, D), jnp.float32) for _ in range(NGB)]
        + [pltpu.SemaphoreType.DMA for _ in range(NCB + 2 * NGB)]
    ),
)


def _dense_body(li_ref, l_ref, e_ref, ws_ref, wd_ref, o_ref):
    x = jnp.dot(li_ref[...], ws_ref[0], preferred_element_type=jnp.float32)
    x = x + jnp.dot(l_ref[...] * e_ref[...], wd_ref[0],
                    preferred_element_type=jnp.float32)
    o_ref[...] = jnp.where(x >= 0, x, 0.2 * x)


_BLK = 1000
_dense = pl.pallas_call(
    _dense_body,
    grid=(N // _BLK,),
    in_specs=[
        pl.BlockSpec((_BLK, D), lambda i: (i, 0)),
        pl.BlockSpec((_BLK, D), lambda i: (i, 0)),
        pl.BlockSpec((_BLK, D), lambda i: (i, 0)),
        pl.BlockSpec((1, D, D), lambda i: (i // (NE // _BLK), 0, 0)),
        pl.BlockSpec((1, D, D), lambda i: (i // (NE // _BLK), 0, 0)),
    ],
    out_specs=pl.BlockSpec((_BLK, D), lambda i: (i, 0)),
    out_shape=jax.ShapeDtypeStruct((N, D), jnp.float32),
)


def kernel(initial_ebs,
           li_rows_user, li_cols_user, li_vals_user,
           l_rows_user, l_cols_user, l_vals_user,
           li_rows_item, li_cols_item, li_vals_item,
           l_rows_item, l_cols_item, l_vals_item,
           w_side_0_user, w_dot_0_user, w_side_0_item, w_dot_0_item,
           w_side_1_user, w_dot_1_user, w_side_1_item, w_dot_1_item,
           cluster_no):
    cols = jnp.stack([li_cols_user, l_cols_user, li_cols_item, l_cols_item])
    rows = jnp.stack([li_rows_user, l_rows_user, li_rows_item, l_rows_item])
    vals = jnp.stack([li_vals_user, l_vals_user, li_vals_item, l_vals_item])
    comb = jnp.stack(
        [cols, rows, lax.bitcast_convert_type(vals, jnp.int32)], axis=1)
    comb = comb.reshape(4, 3, NTILES, NCHUNK, CHUNK)
    comb = comb.transpose(0, 2, 3, 1, 4).reshape(
        2, 2, NTILES, NCHUNK, 3, CHUNK)

    layer_ws = [
        (jnp.stack([w_side_0_user, w_side_0_item]),
         jnp.stack([w_dot_0_user, w_dot_0_item])),
        (jnp.stack([w_side_1_user, w_side_1_item]),
         jnp.stack([w_dot_1_user, w_dot_1_item])),
    ]

    ebs = initial_ebs
    outs = []
    for ws, wd in layer_ws:
        sp = _spmm(ebs, comb)
        li_flat = sp[0].reshape(N, D)
        l_flat = sp[1].reshape(N, D)
        ebs = _dense(li_flat, l_flat, ebs, ws, wd)
        outs.append(ebs)
    return jnp.concatenate(outs, axis=0)


# speedup vs baseline: 1.1319x; 1.1319x over previous
import numpy as np

import jax
import jax.numpy as jnp
from jax import lax
from jax.experimental import pallas as pl
from jax.experimental.pallas import tpu as pltpu
from jax.experimental.pallas import tpu_sc as plsc

N = 10000
D = 128
E = 160000
NE = 5000
NTILES = 16
EPT = E // NTILES
CHUNK = 80
NCHUNK = EPT // CHUNK
ACC_ROWS = 5120
ROWS_PT = ACC_ROWS // NTILES
DW = D // 2

NCB = 6
NGB = 3
NSB = 2
NGRP = CHUNK // 16

_PERM = np.empty((D,), np.int64)
for _v in range(D // 32):
    for _i in range(16):
        _PERM[32 * _v + 2 * _i] = 32 * _v + _i
        _PERM[32 * _v + 2 * _i + 1] = 32 * _v + 16 + _i
_QMAT = np.zeros((D, D), np.float32)
for _k in range(D):
    _QMAT[_PERM[_k], _k] = 1.0


def _zero16():
    return jnp.zeros((16,), jnp.float32)


def _spmm_body(ebs_hbm, comb_hbm, out_hbm,
               acc, cb0, cb1, cb2, cb3, cb4, cb5, gbuf0, gbuf1, gbuf2,
               sbuf0, sbuf1,
               cs0, cs1, cs2, cs3, cs4, cs5, gs0, gs1, gs2, ss0, ss1):
    c = lax.axis_index("c")
    s = lax.axis_index("s")
    row0 = s * ROWS_PT
    cbufs = [cb0, cb1, cb2, cb3, cb4, cb5]
    csems = [cs0, cs1, cs2, cs3, cs4, cs5]
    gbufs = [gbuf0, gbuf1, gbuf2]
    gsems = [gs0, gs1, gs2]
    sbufs = [sbuf0, sbuf1]
    ssems = [ss0, ss1]

    def zrow(r, _):
        for j in range(8):
            sbuf0[r, pl.ds(j * 16, 16)] = _zero16()
        return 0
    lax.fori_loop(0, CHUNK, zrow, 0)
    for m in range(2):
        for q in range(ROWS_PT // CHUNK):
            pltpu.sync_copy(sbuf0, acc.at[m, pl.ds(row0 + q * CHUNK, CHUNK)])
    plsc.subcore_barrier()

    def start_cload(m, k, b):
        pltpu.async_copy(comb_hbm.at[c, m, s, k], cbufs[b], csems[b])

    def wait_cload(b):
        pltpu.make_async_copy(comb_hbm.at[0, 0, 0, 0], cbufs[b], csems[b]).wait()

    def start_gather(b6, b3):
        pltpu.async_copy(ebs_hbm.at[cbufs[b6].at[0]], gbufs[b3], gsems[b3])

    def wait_gather(b3):
        pltpu.make_async_copy(ebs_hbm.at[pl.ds(0, CHUNK)], gbufs[b3],
                              gsems[b3]).wait()

    def start_scatter(m, b2, b6):
        pltpu.async_copy(sbufs[b2], acc.at[m].at[cbufs[b6].at[1]], ssems[b2],
                         add=True)

    def wait_scatter(b2):
        pltpu.make_async_copy(out_hbm.at[0, 0, pl.ds(0, CHUNK)], sbufs[b2],
                              ssems[b2]).wait()

    def scale(b3, b2, b6):
        gb = gbufs[b3]
        sb = sbufs[b2]
        cb = cbufs[b6]

        def grp(g, _):
            vv = plsc.bitcast(cb[2, pl.ds(g * 16, 16)], jnp.float32)
            for t in range(16):
                vb = vv.at[jnp.full((16,), t, jnp.int32)].get(
                    mode='promise_in_bounds')
                r = g * 16 + t
                for v in range(4):
                    w = gb[r, pl.ds(v * 16, 16)]
                    a, b = plsc.unpack(plsc.bitcast(w, jnp.bfloat16),
                                       format=plsc.PackFormat.INTERLEAVED)
                    sb[r, pl.ds(v * 32, 16)] = a * vb
                    sb[r, pl.ds(v * 32 + 16, 16)] = b * vb
            return 0
        lax.fori_loop(0, NGRP, grp, 0)

    def step(m, k, j, maybe_first=False, launch=True, refill=True):
        b3, b2, b6 = j % NGB, j % NSB, j % NCB
        wait_gather(b3)
        if maybe_first:
            @pl.when(k >= NSB)
            def _():
                wait_scatter(b2)
        else:
            wait_scatter(b2)
        if refill:
            start_cload(m, k + NCB - NSB, (j + NCB - NSB) % NCB)
        scale(b3, b2, b6)
        start_scatter(m, b2, b6)
        if launch:
            b3n, b6n = (j + 2) % NGB, (j + 2) % NCB
            wait_cload(b6n)
            start_gather(b6n, b3n)

    def matrix_body(m, _):
        for b in range(NCB - NSB):
            start_cload(m, b, b)
        wait_cload(0)
        start_gather(0, 0)
        wait_cload(1)
        start_gather(1, 1)

        def six(i, _):
            for j in range(NCB):
                step(m, i * NCB + j, j, maybe_first=(j < NSB))
            return 0

        lax.fori_loop(0, (NCHUNK - 5) // NCB, six, 0)
        for j in range(5):
            k = NCHUNK - 5 + j
            step(m, k, k % NCB, launch=(k + 2 < NCHUNK),
                 refill=(k + NCB - NSB < NCHUNK))
        for k in range(NCHUNK - NSB, NCHUNK):
            wait_scatter(k % NSB)
        return 0

    lax.fori_loop(0, 2, matrix_body, 0)

    plsc.subcore_barrier()

    for m in range(2):
        @pl.when(s < NTILES - 1)
        def _():
            pltpu.sync_copy(acc.at[m, pl.ds(row0, ROWS_PT)],
                            out_hbm.at[m, c, pl.ds(row0, ROWS_PT)])

        @pl.when(s == NTILES - 1)
        def _():
            pltpu.sync_copy(acc.at[m, pl.ds(NE - 200, 200)],
                            out_hbm.at[m, c, pl.ds(NE - 200, 200)])


_spmm = pl.kernel(
    _spmm_body,
    out_type=jax.ShapeDtypeStruct((2, 2, NE, D), jnp.float32),
    mesh=plsc.VectorSubcoreMesh(core_axis_name="c", subcore_axis_name="s"),
    compiler_params=pltpu.CompilerParams(needs_layout_passes=False,
                                         use_tc_tiling_on_sc=False),
    scratch_types=(
        [pltpu.VMEM_SHARED((2, ACC_ROWS, D), jnp.float32)]
        + [pltpu.VMEM((3, CHUNK), jnp.int32) for _ in range(NCB)]
        + [pltpu.VMEM((CHUNK, DW), jnp.int32) for _ in range(NGB)]
        + [pltpu.VMEM((CHUNK, D), jnp.float32) for _ in range(NSB)]
        + [pltpu.SemaphoreType.DMA for _ in range(NCB + NGB + NSB)]
    ),
)


def _dense_body(li_ref, l_ref, e_ref, ws_ref, wd_ref, qm_ref, o_ref, op_ref):
    x = jnp.dot(li_ref[...], ws_ref[0], preferred_element_type=jnp.float32)
    x = x + jnp.dot(l_ref[...] * e_ref[...], wd_ref[0],
                    preferred_element_type=jnp.float32)
    x = jnp.where(x >= 0, x, 0.2 * x)
    o_ref[...] = x
    op_ref[...] = jnp.dot(x, qm_ref[...], preferred_element_type=jnp.float32)


_BLK = 1000
_dense = pl.pallas_call(
    _dense_body,
    grid=(N // _BLK,),
    in_specs=[
        pl.BlockSpec((_BLK, D), lambda i: (i, 0)),
        pl.BlockSpec((_BLK, D), lambda i: (i, 0)),
        pl.BlockSpec((_BLK, D), lambda i: (i, 0)),
        pl.BlockSpec((1, D, D), lambda i: (i // (NE // _BLK), 0, 0)),
        pl.BlockSpec((1, D, D), lambda i: (i // (NE // _BLK), 0, 0)),
        pl.BlockSpec((D, D), lambda i: (0, 0)),
    ],
    out_specs=[
        pl.BlockSpec((_BLK, D), lambda i: (i, 0)),
        pl.BlockSpec((_BLK, D), lambda i: (i, 0)),
    ],
    out_shape=[
        jax.ShapeDtypeStruct((N, D), jnp.float32),
        jax.ShapeDtypeStruct((N, D), jnp.float32),
    ],
)


def _pack_table(xp):
    xb = xp.astype(jnp.bfloat16).reshape(N, DW, 2)
    return lax.bitcast_convert_type(xb, jnp.int32)


def kernel(initial_ebs,
           li_rows_user, li_cols_user, li_vals_user,
           l_rows_user, l_cols_user, l_vals_user,
           li_rows_item, li_cols_item, li_vals_item,
           l_rows_item, l_cols_item, l_vals_item,
           w_side_0_user, w_dot_0_user, w_side_0_item, w_dot_0_item,
           w_side_1_user, w_dot_1_user, w_side_1_item, w_dot_1_item,
           cluster_no):
    cols = jnp.stack([li_cols_user, l_cols_user, li_cols_item, l_cols_item])
    rows = jnp.stack([li_rows_user, l_rows_user, li_rows_item, l_rows_item])
    vals = jnp.stack([li_vals_user, l_vals_user, li_vals_item, l_vals_item])
    comb = jnp.stack(
        [cols, rows, lax.bitcast_convert_type(vals, jnp.int32)], axis=1)
    comb = comb.reshape(4, 3, NTILES, NCHUNK, CHUNK)
    comb = comb.transpose(0, 2, 3, 1, 4).reshape(
        2, 2, NTILES, NCHUNK, 3, CHUNK)

    layer_ws = [
        (jnp.stack([w_side_0_user, w_side_0_item]),
         jnp.stack([w_dot_0_user, w_dot_0_item])),
        (jnp.stack([w_side_1_user, w_side_1_item]),
         jnp.stack([w_dot_1_user, w_dot_1_item])),
    ]
    qmat = jnp.asarray(_QMAT)

    ebs = initial_ebs
    ebs_pk = _pack_table(jnp.take(initial_ebs, jnp.asarray(_PERM), axis=1))
    outs = []
    for ws, wd in layer_ws:
        sp = _spmm(ebs_pk, comb)
        li_flat = sp[0].reshape(N, D)
        l_flat = sp[1].reshape(N, D)
        ebs, ebs_p = _dense(li_flat, l_flat, ebs, ws, wd, qmat)
        ebs_pk = _pack_table(ebs_p)
        outs.append(ebs)
    return jnp.concatenate(outs, axis=0)


# speedup vs baseline: 2.3240x; 2.0532x over previous
import jax
import jax.numpy as jnp
from jax import lax
from jax.experimental import pallas as pl
from jax.experimental.pallas import tpu as pltpu
from jax.experimental.pallas import tpu_sc as plsc

N = 10000
D = 128
E = 160000
NE = 5000
NTILES = 16
EPT = E // NTILES
CHUNK = 80
NCHUNK = EPT // CHUNK
ACC_ROWS = 5120
ROWS_PT = ACC_ROWS // NTILES

NCB = 8
NGB = 4
NGRP = CHUNK // 16
NTAIL = NCHUNK - (NCHUNK // NCB) * NCB


def _zero16():
    return jnp.zeros((16,), jnp.float32)


def _spmm_body(ebs_hbm, comb_hbm, out_hbm, acc,
               cb0, cb1, cb2, cb3, cb4, cb5, cb6, cb7,
               gbuf0, gbuf1, gbuf2, gbuf3,
               cs0, cs1, cs2, cs3, cs4, cs5, cs6, cs7,
               gs0, gs1, gs2, gs3, ss0, ss1, ss2, ss3):
    c = lax.axis_index("c")
    s = lax.axis_index("s")
    row0 = s * ROWS_PT
    cbufs = [cb0, cb1, cb2, cb3, cb4, cb5, cb6, cb7]
    csems = [cs0, cs1, cs2, cs3, cs4, cs5, cs6, cs7]
    gbufs = [gbuf0, gbuf1, gbuf2, gbuf3]
    gsems = [gs0, gs1, gs2, gs3]
    ssems = [ss0, ss1, ss2, ss3]

    def zrow(r, _):
        for j in range(8):
            gbuf0[r, pl.ds(j * 16, 16)] = _zero16()
        return 0
    lax.fori_loop(0, CHUNK, zrow, 0)
    for m in range(2):
        for q in range(ROWS_PT // CHUNK):
            pltpu.sync_copy(gbuf0, acc.at[m, pl.ds(row0 + q * CHUNK, CHUNK)])
    plsc.subcore_barrier()

    def start_cload(m, k, b):
        pltpu.async_copy(comb_hbm.at[c, m, s, k], cbufs[b], csems[b])

    def wait_cload(b):
        pltpu.make_async_copy(comb_hbm.at[0, 0, 0, 0], cbufs[b], csems[b]).wait()

    def start_gather(b8, b4):
        pltpu.async_copy(ebs_hbm.at[cbufs[b8].at[0]], gbufs[b4], gsems[b4])

    def wait_gather(b4):
        pltpu.make_async_copy(ebs_hbm.at[pl.ds(0, CHUNK)], gbufs[b4],
                              gsems[b4]).wait()

    def start_scatter(m, b4, b8):
        pltpu.async_copy(gbufs[b4], acc.at[m].at[cbufs[b8].at[1]], ssems[b4],
                         add=True)

    def wait_scatter(b4):
        pltpu.make_async_copy(ebs_hbm.at[pl.ds(0, CHUNK)], gbufs[b4],
                              ssems[b4]).wait()

    def scale(b4, b8):
        buf = gbufs[b4]
        cb = cbufs[b8]

        def grp(g, _):
            vv = plsc.bitcast(cb[2, pl.ds(g * 16, 16)], jnp.float32)
            for t in range(16):
                vb = vv.at[jnp.full((16,), t, jnp.int32)].get(
                    mode='promise_in_bounds')
                r = g * 16 + t
                for jj in range(8):
                    sl = pl.ds(jj * 16, 16)
                    buf[r, sl] = buf[r, sl] * vb
            return 0
        lax.fori_loop(0, NGRP, grp, 0)

    def step(m, k, j, maybe_first=False, launch=True, refill=True,
             guard_refill=False):
        b4, b8 = j % NGB, j % NCB
        wait_gather(b4)
        scale(b4, b8)
        start_scatter(m, b4, b8)
        if launch:
            b4n, b8n = (j + 2) % NGB, (j + 2) % NCB

            def _refill():
                if refill:
                    if guard_refill:
                        @pl.when(k + NCB - 2 < NCHUNK)
                        def _():
                            start_cload(m, k + NCB - 2, (j + NCB - 2) % NCB)
                    else:
                        start_cload(m, k + NCB - 2, (j + NCB - 2) % NCB)

            if maybe_first:
                @pl.when(k >= 2)
                def _():
                    wait_scatter(b4n)
                    _refill()

                @pl.when(k < 2)
                def _():
                    _refill()
            else:
                wait_scatter(b4n)
                _refill()
            wait_cload(b8n)
            start_gather(b8n, b4n)

    def matrix_body(m, _):
        for b in range(NCB - 2):
            start_cload(m, b, b)
        wait_cload(0)
        start_gather(0, 0)
        wait_cload(1)
        start_gather(1, 1)

        def octo(i, _):
            for j in range(NCB):
                step(m, i * NCB + j, j, maybe_first=(j < 2),
                     guard_refill=(j == NCB - 1))
            return 0

        lax.fori_loop(0, NCHUNK // NCB, octo, 0)
        for j in range(NTAIL):
            k = NCHUNK - NTAIL + j
            step(m, k, k % NCB, launch=(k + 2 < NCHUNK),
                 refill=(k + NCB - 2 < NCHUNK))
        for k in range(NCHUNK - NGB, NCHUNK):
            wait_scatter(k % NGB)
        return 0

    lax.fori_loop(0, 2, matrix_body, 0)

    plsc.subcore_barrier()

    for m in range(2):
        @pl.when(s < NTILES - 1)
        def _():
            pltpu.sync_copy(acc.at[m, pl.ds(row0, ROWS_PT)],
                            out_hbm.at[m, c, pl.ds(row0, ROWS_PT)])

        @pl.when(s == NTILES - 1)
        def _():
            pltpu.sync_copy(acc.at[m, pl.ds(NE - 200, 200)],
                            out_hbm.at[m, c, pl.ds(NE - 200, 200)])


_spmm = pl.kernel(
    _spmm_body,
    out_type=jax.ShapeDtypeStruct((2, 2, NE, D), jnp.float32),
    mesh=plsc.VectorSubcoreMesh(core_axis_name="c", subcore_axis_name="s"),
    compiler_params=pltpu.CompilerParams(needs_layout_passes=False),
    scratch_types=(
        [pltpu.VMEM_SHARED((2, ACC_ROWS, D), jnp.float32)]
        + [pltpu.VMEM((3, CHUNK), jnp.int32) for _ in range(NCB)]
        + [pltpu.VMEM((CHUNK, D), jnp.float32) for _ in range(NGB)]
        + [pltpu.SemaphoreType.DMA for _ in range(NCB + 2 * NGB)]
    ),
)


def _dense_body(li_ref, l_ref, e_ref, ws_ref, wd_ref, o_ref):
    x = jnp.dot(li_ref[...], ws_ref[0], preferred_element_type=jnp.float32)
    x = x + jnp.dot(l_ref[...] * e_ref[...], wd_ref[0],
                    preferred_element_type=jnp.float32)
    o_ref[...] = jnp.where(x >= 0, x, 0.2 * x)


_BLK = 1000
_dense = pl.pallas_call(
    _dense_body,
    grid=(N // _BLK,),
    in_specs=[
        pl.BlockSpec((_BLK, D), lambda i: (i, 0)),
        pl.BlockSpec((_BLK, D), lambda i: (i, 0)),
        pl.BlockSpec((_BLK, D), lambda i: (i, 0)),
        pl.BlockSpec((1, D, D), lambda i: (i // (NE // _BLK), 0, 0)),
        pl.BlockSpec((1, D, D), lambda i: (i // (NE // _BLK), 0, 0)),
    ],
    out_specs=pl.BlockSpec((_BLK, D), lambda i: (i, 0)),
    out_shape=jax.ShapeDtypeStruct((N, D), jnp.float32),
)


def kernel(initial_ebs,
           li_rows_user, li_cols_user, li_vals_user,
           l_rows_user, l_cols_user, l_vals_user,
           li_rows_item, li_cols_item, li_vals_item,
           l_rows_item, l_cols_item, l_vals_item,
           w_side_0_user, w_dot_0_user, w_side_0_item, w_dot_0_item,
           w_side_1_user, w_dot_1_user, w_side_1_item, w_dot_1_item,
           cluster_no):
    cols = jnp.stack([li_cols_user, l_cols_user, li_cols_item, l_cols_item])
    rows = jnp.stack([li_rows_user, l_rows_user, li_rows_item, l_rows_item])
    vals = jnp.stack([li_vals_user, l_vals_user, li_vals_item, l_vals_item])
    comb = jnp.stack(
        [cols, rows, lax.bitcast_convert_type(vals, jnp.int32)], axis=1)
    comb = comb.reshape(4, 3, NTILES, NCHUNK, CHUNK)
    comb = comb.transpose(0, 2, 3, 1, 4).reshape(
        2, 2, NTILES, NCHUNK, 3, CHUNK)

    layer_ws = [
        (jnp.stack([w_side_0_user, w_side_0_item]),
         jnp.stack([w_dot_0_user, w_dot_0_item])),
        (jnp.stack([w_side_1_user, w_side_1_item]),
         jnp.stack([w_dot_1_user, w_dot_1_item])),
    ]

    ebs = initial_ebs
    outs = []
    for ws, wd in layer_ws:
        sp = _spmm(ebs, comb)
        li_flat = sp[0].reshape(N, D)
        l_flat = sp[1].reshape(N, D)
        ebs = _dense(li_flat, l_flat, ebs, ws, wd)
        outs.append(ebs)
    return jnp.concatenate(outs, axis=0)


# speedup vs baseline: 2.3707x; 1.0201x over previous
import jax
import jax.numpy as jnp
from jax import lax
from jax.experimental import pallas as pl
from jax.experimental.pallas import tpu as pltpu
from jax.experimental.pallas import tpu_sc as plsc

N = 10000
D = 128
E = 160000
NE = 5000
NTILES = 16
EPT = E // NTILES
CHUNK = 80
NCHUNK = EPT // CHUNK
ACC_ROWS = 5120
ROWS_PT = ACC_ROWS // NTILES

NCB = 8
NGB = 4
NGRP = CHUNK // 16
NTAIL = NCHUNK - (NCHUNK // NCB) * NCB


def _zero16():
    return jnp.zeros((16,), jnp.float32)


def _spmm_body(ebs_hbm, comb_hbm, out_hbm, acc,
               cb0, cb1, cb2, cb3, cb4, cb5, cb6, cb7,
               gbuf0, gbuf1, gbuf2, gbuf3,
               cs0, cs1, cs2, cs3, cs4, cs5, cs6, cs7,
               gs0, gs1, gs2, gs3, ss0, ss1, ss2, ss3):
    c = lax.axis_index("c")
    s = lax.axis_index("s")
    row0 = s * ROWS_PT
    cbufs = [cb0, cb1, cb2, cb3, cb4, cb5, cb6, cb7]
    csems = [cs0, cs1, cs2, cs3, cs4, cs5, cs6, cs7]
    gbufs = [gbuf0, gbuf1, gbuf2, gbuf3]
    gsems = [gs0, gs1, gs2, gs3]
    ssems = [ss0, ss1, ss2, ss3]

    def zrow(r, _):
        for j in range(8):
            gbuf0[r, pl.ds(j * 16, 16)] = _zero16()
        return 0
    lax.fori_loop(0, CHUNK, zrow, 0)
    for m in range(2):
        for q in range(ROWS_PT // CHUNK):
            pltpu.sync_copy(gbuf0, acc.at[m, pl.ds(row0 + q * CHUNK, CHUNK)])
    plsc.subcore_barrier()

    def start_cload(m, k, b):
        pltpu.async_copy(comb_hbm.at[c, m, s, k], cbufs[b], csems[b])

    def wait_cload(b):
        pltpu.make_async_copy(comb_hbm.at[0, 0, 0, 0], cbufs[b], csems[b]).wait()

    def start_gather(b8, b4):
        pltpu.async_copy(ebs_hbm.at[cbufs[b8].at[0]], gbufs[b4], gsems[b4])

    def wait_gather(b4):
        pltpu.make_async_copy(ebs_hbm.at[pl.ds(0, CHUNK)], gbufs[b4],
                              gsems[b4]).wait()

    def start_scatter(m, b4, b8):
        pltpu.async_copy(gbufs[b4], acc.at[m].at[cbufs[b8].at[1]], ssems[b4],
                         add=True)

    def wait_scatter(b4):
        pltpu.make_async_copy(ebs_hbm.at[pl.ds(0, CHUNK)], gbufs[b4],
                              ssems[b4]).wait()

    def scale(b4, b8):
        buf = gbufs[b4]
        cb = cbufs[b8]

        def grp(g, _):
            vv = plsc.bitcast(cb[2, pl.ds(g * 16, 16)], jnp.float32)
            for t in range(16):
                vb = vv.at[jnp.full((16,), t, jnp.int32)].get(
                    mode='promise_in_bounds')
                r = g * 16 + t
                for jj in range(8):
                    sl = pl.ds(jj * 16, 16)
                    buf[r, sl] = buf[r, sl] * vb
            return 0
        lax.fori_loop(0, NGRP, grp, 0)

    def step(m, k, j, maybe_first=False, launch=True, refill=True,
             guard_refill=False):
        b4, b8 = j % NGB, j % NCB
        wait_gather(b4)
        if launch:
            b4n, b8n = (j + 2) % NGB, (j + 2) % NCB

            def _refill():
                if refill:
                    if guard_refill:
                        @pl.when(k + NCB - 2 < NCHUNK)
                        def _():
                            start_cload(m, k + NCB - 2, (j + NCB - 2) % NCB)
                    else:
                        start_cload(m, k + NCB - 2, (j + NCB - 2) % NCB)

            if maybe_first:
                @pl.when(k >= 2)
                def _():
                    wait_scatter(b4n)
                    _refill()

                @pl.when(k < 2)
                def _():
                    _refill()
            else:
                wait_scatter(b4n)
                _refill()
            wait_cload(b8n)
            start_gather(b8n, b4n)
        scale(b4, b8)
        start_scatter(m, b4, b8)

    def matrix_body(m, _):
        for b in range(NCB - 2):
            start_cload(m, b, b)
        wait_cload(0)
        start_gather(0, 0)
        wait_cload(1)
        start_gather(1, 1)

        def octo(i, _):
            for j in range(NCB):
                step(m, i * NCB + j, j, maybe_first=(j < 2),
                     guard_refill=(j == NCB - 1))
            return 0

        lax.fori_loop(0, NCHUNK // NCB, octo, 0)
        for j in range(NTAIL):
            k = NCHUNK - NTAIL + j
            step(m, k, k % NCB, launch=(k + 2 < NCHUNK),
                 refill=(k + NCB - 2 < NCHUNK))
        for k in range(NCHUNK - NGB, NCHUNK):
            wait_scatter(k % NGB)
        return 0

    lax.fori_loop(0, 2, matrix_body, 0)

    plsc.subcore_barrier()

    for m in range(2):
        @pl.when(s < NTILES - 1)
        def _():
            pltpu.sync_copy(acc.at[m, pl.ds(row0, ROWS_PT)],
                            out_hbm.at[m, c, pl.ds(row0, ROWS_PT)])

        @pl.when(s == NTILES - 1)
        def _():
            pltpu.sync_copy(acc.at[m, pl.ds(NE - 200, 200)],
                            out_hbm.at[m, c, pl.ds(NE - 200, 200)])


_spmm = pl.kernel(
    _spmm_body,
    out_type=jax.ShapeDtypeStruct((2, 2, NE, D), jnp.float32),
    mesh=plsc.VectorSubcoreMesh(core_axis_name="c", subcore_axis_name="s"),
    compiler_params=pltpu.CompilerParams(needs_layout_passes=False),
    scratch_types=(
        [pltpu.VMEM_SHARED((2, ACC_ROWS, D), jnp.float32)]
        + [pltpu.VMEM((3, CHUNK), jnp.int32) for _ in range(NCB)]
        + [pltpu.VMEM((CHUNK, D), jnp.float32) for _ in range(NGB)]
        + [pltpu.SemaphoreType.DMA for _ in range(NCB + 2 * NGB)]
    ),
)


def _dense_body(li_ref, l_ref, e_ref, ws_ref, wd_ref, o_ref):
    x = jnp.dot(li_ref[...], ws_ref[0], preferred_element_type=jnp.float32)
    x = x + jnp.dot(l_ref[...] * e_ref[...], wd_ref[0],
                    preferred_element_type=jnp.float32)
    o_ref[...] = jnp.where(x >= 0, x, 0.2 * x)


_BLK = 1000
_dense = pl.pallas_call(
    _dense_body,
    grid=(N // _BLK,),
    in_specs=[
        pl.BlockSpec((_BLK, D), lambda i: (i, 0)),
        pl.BlockSpec((_BLK, D), lambda i: (i, 0)),
        pl.BlockSpec((_BLK, D), lambda i: (i, 0)),
        pl.BlockSpec((1, D, D), lambda i: (i // (NE // _BLK), 0, 0)),
        pl.BlockSpec((1, D, D), lambda i: (i // (NE // _BLK), 0, 0)),
    ],
    out_specs=pl.BlockSpec((_BLK, D), lambda i: (i, 0)),
    out_shape=jax.ShapeDtypeStruct((N, D), jnp.float32),
)


def kernel(initial_ebs,
           li_rows_user, li_cols_user, li_vals_user,
           l_rows_user, l_cols_user, l_vals_user,
           li_rows_item, li_cols_item, li_vals_item,
           l_rows_item, l_cols_item, l_vals_item,
           w_side_0_user, w_dot_0_user, w_side_0_item, w_dot_0_item,
           w_side_1_user, w_dot_1_user, w_side_1_item, w_dot_1_item,
           cluster_no):
    cols = jnp.stack([li_cols_user, l_cols_user, li_cols_item, l_cols_item])
    rows = jnp.stack([li_rows_user, l_rows_user, li_rows_item, l_rows_item])
    vals = jnp.stack([li_vals_user, l_vals_user, li_vals_item, l_vals_item])
    comb = jnp.stack(
        [cols, rows, lax.bitcast_convert_type(vals, jnp.int32)], axis=1)
    comb = comb.reshape(4, 3, NTILES, NCHUNK, CHUNK)
    comb = comb.transpose(0, 2, 3, 1, 4).reshape(
        2, 2, NTILES, NCHUNK, 3, CHUNK)

    layer_ws = [
        (jnp.stack([w_side_0_user, w_side_0_item]),
         jnp.stack([w_dot_0_user, w_dot_0_item])),
        (jnp.stack([w_side_1_user, w_side_1_item]),
         jnp.stack([w_dot_1_user, w_dot_1_item])),
    ]

    ebs = initial_ebs
    outs = []
    for ws, wd in layer_ws:
        sp = _spmm(ebs, comb)
        li_flat = sp[0].reshape(N, D)
        l_flat = sp[1].reshape(N, D)
        ebs = _dense(li_flat, l_flat, ebs, ws, wd)
        outs.append(ebs)
    return jnp.concatenate(outs, axis=0)


# speedup vs baseline: 2.4465x; 1.0320x over previous
import jax
import jax.numpy as jnp
from jax import lax
from jax.experimental import pallas as pl
from jax.experimental.pallas import tpu as pltpu
from jax.experimental.pallas import tpu_sc as plsc

N = 10000
D = 128
E = 160000
NE = 5000
NTILES = 16
EPT = E // NTILES
CHUNK = 80
NCHUNK = EPT // CHUNK
ACC_ROWS = 5120
ROWS_PT = ACC_ROWS // NTILES

NCB = 8
NGB = 4
NGRP = CHUNK // 16
NTAIL = NCHUNK - (NCHUNK // NCB) * NCB


def _zero16():
    return jnp.zeros((16,), jnp.float32)


def _spmm_body(ebs_hbm, comb_hbm, out_hbm, acc,
               cb0, cb1, cb2, cb3, cb4, cb5, cb6, cb7,
               gbuf0, gbuf1, gbuf2, gbuf3,
               cs0, cs1, cs2, cs3, cs4, cs5, cs6, cs7,
               gs0, gs1, gs2, gs3, ss0, ss1, ss2, ss3):
    c = lax.axis_index("c")
    s = lax.axis_index("s")
    row0 = s * ROWS_PT
    cbufs = [cb0, cb1, cb2, cb3, cb4, cb5, cb6, cb7]
    csems = [cs0, cs1, cs2, cs3, cs4, cs5, cs6, cs7]
    gbufs = [gbuf0, gbuf1, gbuf2, gbuf3]
    gsems = [gs0, gs1, gs2, gs3]
    ssems = [ss0, ss1, ss2, ss3]

    def zrow(r, _):
        for j in range(8):
            gbuf0[r, pl.ds(j * 16, 16)] = _zero16()
        return 0
    lax.fori_loop(0, CHUNK, zrow, 0)
    for m in range(2):
        for q in range(ROWS_PT // CHUNK):
            pltpu.sync_copy(gbuf0, acc.at[m, pl.ds(row0 + q * CHUNK, CHUNK)])
    plsc.subcore_barrier()

    def start_cload(m, k, b):
        pltpu.async_copy(comb_hbm.at[c, m, s, k], cbufs[b], csems[b])

    def wait_cload(b):
        pltpu.make_async_copy(comb_hbm.at[0, 0, 0, 0], cbufs[b], csems[b]).wait()

    def start_gather(b8, b4):
        pltpu.async_copy(ebs_hbm.at[cbufs[b8].at[0]], gbufs[b4], gsems[b4])

    def wait_gather(b4):
        pltpu.make_async_copy(ebs_hbm.at[pl.ds(0, CHUNK)], gbufs[b4],
                              gsems[b4]).wait()

    def start_scatter(m, b4, b8):
        pltpu.async_copy(gbufs[b4], acc.at[m].at[cbufs[b8].at[1]], ssems[b4],
                         add=True)

    def wait_scatter(b4):
        pltpu.make_async_copy(ebs_hbm.at[pl.ds(0, CHUNK)], gbufs[b4],
                              ssems[b4]).wait()

    def scale(b4, b8):
        buf = gbufs[b4]
        cb = cbufs[b8]

        def grp(g, _):
            vv = plsc.bitcast(cb[2, pl.ds(g * 16, 16)], jnp.float32)
            for t in range(16):
                vb = vv.at[jnp.full((16,), t, jnp.int32)].get(
                    mode='promise_in_bounds')
                r = g * 16 + t
                for jj in range(8):
                    sl = pl.ds(jj * 16, 16)
                    buf[r, sl] = buf[r, sl] * vb
            return 0
        lax.fori_loop(0, NGRP, grp, 0)

    def step(m, k, j, maybe_first=False, launch=True, refill=True,
             guard_refill=False):
        b4, b8 = j % NGB, j % NCB
        if launch:
            b4n, b8n = (j + 2) % NGB, (j + 2) % NCB

            def _refill():
                if refill:
                    if guard_refill:
                        @pl.when(k + NCB - 2 < NCHUNK)
                        def _():
                            start_cload(m, k + NCB - 2, (j + NCB - 2) % NCB)
                    else:
                        start_cload(m, k + NCB - 2, (j + NCB - 2) % NCB)

            if maybe_first:
                @pl.when(k >= 2)
                def _():
                    wait_scatter(b4n)
                    _refill()

                @pl.when(k < 2)
                def _():
                    _refill()
            else:
                wait_scatter(b4n)
                _refill()
            wait_cload(b8n)
            start_gather(b8n, b4n)
        wait_gather(b4)
        scale(b4, b8)
        start_scatter(m, b4, b8)

    def matrix_body(m, _):
        for b in range(NCB - 2):
            start_cload(m, b, b)
        wait_cload(0)
        start_gather(0, 0)
        wait_cload(1)
        start_gather(1, 1)

        def octo(i, _):
            for j in range(NCB):
                step(m, i * NCB + j, j, maybe_first=(j < 2),
                     guard_refill=(j == NCB - 1))
            return 0

        lax.fori_loop(0, NCHUNK // NCB, octo, 0)
        for j in range(NTAIL):
            k = NCHUNK - NTAIL + j
            step(m, k, k % NCB, launch=(k + 2 < NCHUNK),
                 refill=(k + NCB - 2 < NCHUNK))
        for k in range(NCHUNK - NGB, NCHUNK):
            wait_scatter(k % NGB)
        return 0

    lax.fori_loop(0, 2, matrix_body, 0)

    plsc.subcore_barrier()

    for m in range(2):
        @pl.when(s < NTILES - 1)
        def _():
            pltpu.sync_copy(acc.at[m, pl.ds(row0, ROWS_PT)],
                            out_hbm.at[m, c, pl.ds(row0, ROWS_PT)])

        @pl.when(s == NTILES - 1)
        def _():
            pltpu.sync_copy(acc.at[m, pl.ds(NE - 200, 200)],
                            out_hbm.at[m, c, pl.ds(NE - 200, 200)])


_spmm = pl.kernel(
    _spmm_body,
    out_type=jax.ShapeDtypeStruct((2, 2, NE, D), jnp.float32),
    mesh=plsc.VectorSubcoreMesh(core_axis_name="c", subcore_axis_name="s"),
    compiler_params=pltpu.CompilerParams(needs_layout_passes=False),
    scratch_types=(
        [pltpu.VMEM_SHARED((2, ACC_ROWS, D), jnp.float32)]
        + [pltpu.VMEM((3, CHUNK), jnp.int32) for _ in range(NCB)]
        + [pltpu.VMEM((CHUNK, D), jnp.float32) for _ in range(NGB)]
        + [pltpu.SemaphoreType.DMA for _ in range(NCB + 2 * NGB)]
    ),
)


def _dense_body(li_ref, l_ref, e_ref, ws_ref, wd_ref, o_ref):
    x = jnp.dot(li_ref[...], ws_ref[0], preferred_element_type=jnp.float32)
    x = x + jnp.dot(l_ref[...] * e_ref[...], wd_ref[0],
                    preferred_element_type=jnp.float32)
    o_ref[...] = jnp.where(x >= 0, x, 0.2 * x)


_BLK = 1000
_dense = pl.pallas_call(
    _dense_body,
    grid=(N // _BLK,),
    in_specs=[
        pl.BlockSpec((_BLK, D), lambda i: (i, 0)),
        pl.BlockSpec((_BLK, D), lambda i: (i, 0)),
        pl.BlockSpec((_BLK, D), lambda i: (i, 0)),
        pl.BlockSpec((1, D, D), lambda i: (i // (NE // _BLK), 0, 0)),
        pl.BlockSpec((1, D, D), lambda i: (i // (NE // _BLK), 0, 0)),
    ],
    out_specs=pl.BlockSpec((_BLK, D), lambda i: (i, 0)),
    out_shape=jax.ShapeDtypeStruct((N, D), jnp.float32),
)


def kernel(initial_ebs,
           li_rows_user, li_cols_user, li_vals_user,
           l_rows_user, l_cols_user, l_vals_user,
           li_rows_item, li_cols_item, li_vals_item,
           l_rows_item, l_cols_item, l_vals_item,
           w_side_0_user, w_dot_0_user, w_side_0_item, w_dot_0_item,
           w_side_1_user, w_dot_1_user, w_side_1_item, w_dot_1_item,
           cluster_no):
    cols = jnp.stack([li_cols_user, l_cols_user, li_cols_item, l_cols_item])
    rows = jnp.stack([li_rows_user, l_rows_user, li_rows_item, l_rows_item])
    vals = jnp.stack([li_vals_user, l_vals_user, li_vals_item, l_vals_item])
    comb = jnp.stack(
        [cols, rows, lax.bitcast_convert_type(vals, jnp.int32)], axis=1)
    comb = comb.reshape(4, 3, NTILES, NCHUNK, CHUNK)
    comb = comb.transpose(0, 2, 3, 1, 4).reshape(
        2, 2, NTILES, NCHUNK, 3, CHUNK)

    layer_ws = [
        (jnp.stack([w_side_0_user, w_side_0_item]),
         jnp.stack([w_dot_0_user, w_dot_0_item])),
        (jnp.stack([w_side_1_user, w_side_1_item]),
         jnp.stack([w_dot_1_user, w_dot_1_item])),
    ]

    ebs = initial_ebs
    outs = []
    for ws, wd in layer_ws:
        sp = _spmm(ebs, comb)
        li_flat = sp[0].reshape(N, D)
        l_flat = sp[1].reshape(N, D)
        ebs = _dense(li_flat, l_flat, ebs, ws, wd)
        outs.append(ebs)
    return jnp.concatenate(outs, axis=0)
